# baseline (speedup 1.0000x reference)
import jax
import jax.numpy as jnp
from jax.experimental import pallas as pl
from jax.experimental.pallas import tpu as pltpu

_BT = 256


def _fused_body(x_ref, a1_ref, a2_ref, sh1_ref, sh2_ref,
                wfc1_ref, bfc1_ref, wfc2_ref, bfc2_ref, o_ref, y1_ref):
    bf = jnp.bfloat16
    sh1 = sh1_ref[...]
    sh2 = sh2_ref[...]

    def conv1_row(h):
        lo, hi = max(h - 1, 0), min(h + 1, 31)
        k0 = 128 * (lo - h + 1)
        return jnp.dot(x_ref[:, 128 * lo:128 * (hi + 1)],
                       a1_ref[k0:k0 + 128 * (hi - lo + 1), :],
                       preferred_element_type=jnp.float32)

    for k in range(16):
        m = jnp.maximum(conv1_row(2 * k), conv1_row(2 * k + 1))
        m = jnp.maximum(m[:, 0:256], m[:, 256:512])
        y1_ref[:, 256 * k:256 * k + 256] = (
            jnp.maximum(m + sh1, 0.0).astype(bf))

    def conv2_row(h):
        lo, hi = max(h - 1, 0), min(h + 1, 15)
        k0 = 256 * (lo - h + 1)
        return jnp.dot(y1_ref[:, 256 * lo:256 * (hi + 1)],
                       a2_ref[k0:k0 + 256 * (hi - lo + 1), :],
                       preferred_element_type=jnp.float32)

    y2 = []
    for r in range(8):
        m = jnp.maximum(conv2_row(2 * r), conv2_row(2 * r + 1))
        m = jnp.maximum(m[:, 0:256], m[:, 256:512])
        y2.append(jnp.maximum(m + sh2, 0.0))

    flat = jnp.concatenate(y2, axis=1)

    h = jnp.maximum(
        jnp.dot(flat, wfc1_ref[...],
                preferred_element_type=jnp.float32)[:, 0:64]
        + bfc1_ref[...], 0.0)
    z = jnp.sum(h * wfc2_ref[...], axis=-1, keepdims=True) + bfc2_ref[...]
    o_ref[...] = 1.0 / (1.0 + jnp.exp(-z))


def kernel(x_nchw, a1e, a1o, a2e, a2o, sc1, sh1, sc2, sh2,
           wfc1, bfc1, wfc2, bfc2):
    n = x_nchw.shape[0]
    bt = min(_BT, n)
    bf = jnp.bfloat16

    x_t = jnp.pad(jnp.transpose(x_nchw.astype(bf),
                                (0, 2, 1, 3)).reshape(n, 32, 96),
                  ((0, 0), (0, 0), (0, 32))).reshape(n, 4096)

    def stack1(a):
        return a.reshape(3, 32, 3, 256).transpose(0, 2, 1, 3).reshape(3, 96, 256)

    a1 = (jnp.pad(jnp.concatenate([stack1(a1e), stack1(a1o)], axis=2),
                  ((0, 0), (0, 32), (0, 0)))
          .reshape(384, 512)
          * jnp.concatenate([sc1, sc1], axis=1)).astype(bf)
    a2 = (jnp.concatenate([a2e.reshape(768, 256), a2o.reshape(768, 256)],
                          axis=1)
          * jnp.concatenate([sc2, sc2], axis=1)).astype(bf)

    wfc1p = jnp.pad(wfc1, ((0, 0), (0, 192)))

    c2 = lambda i: (0, 0)
    out = pl.pallas_call(
        _fused_body,
        out_shape=jax.ShapeDtypeStruct((n, 1), jnp.float32),
        grid=(n // bt,),
        in_specs=[
            pl.BlockSpec((bt, 4096), lambda i: (i, 0)),
            pl.BlockSpec((384, 512), c2),
            pl.BlockSpec((768, 512), c2),
            pl.BlockSpec((1, 256), c2),
            pl.BlockSpec((1, 256), c2),
            pl.BlockSpec((2048, 256), c2),
            pl.BlockSpec((1, 64), c2),
            pl.BlockSpec((1, 64), c2),
            pl.BlockSpec((1, 1), c2),
        ],
        out_specs=pl.BlockSpec((bt, 1), lambda i: (i, 0)),
        scratch_shapes=[pltpu.VMEM((bt, 4096), bf)],
        compiler_params=pltpu.CompilerParams(
            dimension_semantics=("arbitrary",)),
    )(x_t, a1, a2, sh1, sh2, wfc1p, bfc1, wfc2, bfc2)
    return out


# speedup vs baseline: 1.0171x; 1.0171x over previous
import jax
import jax.numpy as jnp
from jax.experimental import pallas as pl
from jax.experimental.pallas import tpu as pltpu

_BT = 512


def _fused_body(x_ref, a1_ref, a2_ref, sh1_ref, sh2_ref,
                wfc1_ref, bfc1_ref, wfc2_ref, bfc2_ref, o_ref, y1_ref):
    bf = jnp.bfloat16
    sh1 = sh1_ref[...]
    sh2 = sh2_ref[...]

    def conv1_row(h):
        lo, hi = max(h - 1, 0), min(h + 1, 31)
        k0 = 128 * (lo - h + 1)
        return jnp.dot(x_ref[:, 128 * lo:128 * (hi + 1)],
                       a1_ref[k0:k0 + 128 * (hi - lo + 1), :],
                       preferred_element_type=jnp.float32)

    for k in range(16):
        m = jnp.maximum(conv1_row(2 * k), conv1_row(2 * k + 1))
        m = jnp.maximum(m[:, 0:256], m[:, 256:512])
        y1_ref[:, 256 * k:256 * k + 256] = (
            jnp.maximum(m + sh1, 0.0).astype(bf))

    def conv2_row(h):
        lo, hi = max(h - 1, 0), min(h + 1, 15)
        k0 = 256 * (lo - h + 1)
        return jnp.dot(y1_ref[:, 256 * lo:256 * (hi + 1)],
                       a2_ref[k0:k0 + 256 * (hi - lo + 1), :],
                       preferred_element_type=jnp.float32)

    y2 = []
    for r in range(8):
        m = jnp.maximum(conv2_row(2 * r), conv2_row(2 * r + 1))
        m = jnp.maximum(m[:, 0:256], m[:, 256:512])
        y2.append(jnp.maximum(m + sh2, 0.0))

    flat = jnp.concatenate(y2, axis=1)

    h = jnp.maximum(
        jnp.dot(flat, wfc1_ref[...],
                preferred_element_type=jnp.float32)[:, 0:64]
        + bfc1_ref[...], 0.0)
    z = jnp.sum(h * wfc2_ref[...], axis=-1, keepdims=True) + bfc2_ref[...]
    o_ref[...] = 1.0 / (1.0 + jnp.exp(-z))


def kernel(x_nchw, a1e, a1o, a2e, a2o, sc1, sh1, sc2, sh2,
           wfc1, bfc1, wfc2, bfc2):
    n = x_nchw.shape[0]
    bt = min(_BT, n)
    bf = jnp.bfloat16

    x_t = jnp.transpose(
        jnp.pad(x_nchw, ((0, 0), (0, 1), (0, 0), (0, 0))).astype(bf),
        (0, 2, 1, 3)).reshape(n, 4096)

    def stack1(a):
        return a.reshape(3, 32, 3, 256).transpose(0, 2, 1, 3).reshape(3, 96, 256)

    a1 = (jnp.pad(jnp.concatenate([stack1(a1e), stack1(a1o)], axis=2),
                  ((0, 0), (0, 32), (0, 0)))
          .reshape(384, 512)
          * jnp.concatenate([sc1, sc1], axis=1)).astype(bf)
    a2 = (jnp.concatenate([a2e.reshape(768, 256), a2o.reshape(768, 256)],
                          axis=1)
          * jnp.concatenate([sc2, sc2], axis=1)).astype(bf)

    wfc1p = jnp.pad(wfc1, ((0, 0), (0, 192)))

    c2 = lambda i: (0, 0)
    out = pl.pallas_call(
        _fused_body,
        out_shape=jax.ShapeDtypeStruct((n, 1), jnp.float32),
        grid=(n // bt,),
        in_specs=[
            pl.BlockSpec((bt, 4096), lambda i: (i, 0)),
            pl.BlockSpec((384, 512), c2),
            pl.BlockSpec((768, 512), c2),
            pl.BlockSpec((1, 256), c2),
            pl.BlockSpec((1, 256), c2),
            pl.BlockSpec((2048, 256), c2),
            pl.BlockSpec((1, 64), c2),
            pl.BlockSpec((1, 64), c2),
            pl.BlockSpec((1, 1), c2),
        ],
        out_specs=pl.BlockSpec((bt, 1), lambda i: (i, 0)),
        scratch_shapes=[pltpu.VMEM((bt, 4096), bf)],
        compiler_params=pltpu.CompilerParams(
            dimension_semantics=("arbitrary",)),
    )(x_t, a1, a2, sh1, sh2, wfc1p, bfc1, wfc2, bfc2)
    return out


# speedup vs baseline: 1.0895x; 1.0712x over previous
import jax
import jax.numpy as jnp
from jax.experimental import pallas as pl
from jax.experimental.pallas import tpu as pltpu

_BT = 512


def _fused_body(x_ref, a1_ref, a2_ref, sh1_ref, sh2_ref,
                wfc1_ref, bfc1_ref, wfc2_ref, bfc2_ref, o_ref, y1_ref):
    bf = jnp.bfloat16
    sh1 = sh1_ref[...]
    sh2 = sh2_ref[...]

    def conv1_row(h):
        lo, hi = max(h - 1, 0), min(h + 1, 31)
        k0 = 96 * (lo - h + 1)
        return jnp.dot(x_ref[:, 96 * lo:96 * (hi + 1)],
                       a1_ref[k0:k0 + 96 * (hi - lo + 1), :],
                       preferred_element_type=jnp.float32)

    for k in range(16):
        m = jnp.maximum(conv1_row(2 * k), conv1_row(2 * k + 1))
        m = jnp.maximum(m[:, 0:256], m[:, 256:512])
        y1_ref[:, 256 * k:256 * k + 256] = (
            jnp.maximum(m + sh1, 0.0).astype(bf))

    def conv2_row(h):
        lo, hi = max(h - 1, 0), min(h + 1, 15)
        k0 = 256 * (lo - h + 1)
        return jnp.dot(y1_ref[:, 256 * lo:256 * (hi + 1)],
                       a2_ref[k0:k0 + 256 * (hi - lo + 1), :],
                       preferred_element_type=jnp.float32)

    y2 = []
    for r in range(8):
        m = jnp.maximum(conv2_row(2 * r), conv2_row(2 * r + 1))
        m = jnp.maximum(m[:, 0:256], m[:, 256:512])
        y2.append(jnp.maximum(m + sh2, 0.0))

    flat = jnp.concatenate(y2, axis=1)

    h = jnp.maximum(
        jnp.dot(flat, wfc1_ref[...],
                preferred_element_type=jnp.float32)[:, 0:64]
        + bfc1_ref[...], 0.0)
    z = jnp.sum(h * wfc2_ref[...], axis=-1, keepdims=True) + bfc2_ref[...]
    o_ref[...] = 1.0 / (1.0 + jnp.exp(-z))


def kernel(x_nchw, a1e, a1o, a2e, a2o, sc1, sh1, sc2, sh2,
           wfc1, bfc1, wfc2, bfc2):
    n = x_nchw.shape[0]
    bt = min(_BT, n)
    bf = jnp.bfloat16

    x_t = jnp.transpose(x_nchw.astype(bf), (0, 2, 1, 3)).reshape(n, 3072)

    def stack1(a):
        return a.reshape(3, 32, 3, 256).transpose(0, 2, 1, 3).reshape(3, 96, 256)

    a1 = (jnp.concatenate([stack1(a1e), stack1(a1o)], axis=2).reshape(288, 512)
          * jnp.concatenate([sc1, sc1], axis=1)).astype(bf)
    a2 = (jnp.concatenate([a2e.reshape(768, 256), a2o.reshape(768, 256)],
                          axis=1)
          * jnp.concatenate([sc2, sc2], axis=1)).astype(bf)

    wfc1p = jnp.pad(wfc1, ((0, 0), (0, 192)))

    c2 = lambda i: (0, 0)
    out = pl.pallas_call(
        _fused_body,
        out_shape=jax.ShapeDtypeStruct((n, 1), jnp.float32),
        grid=(n // bt,),
        in_specs=[
            pl.BlockSpec((bt, 3072), lambda i: (i, 0)),
            pl.BlockSpec((288, 512), c2),
            pl.BlockSpec((768, 512), c2),
            pl.BlockSpec((1, 256), c2),
            pl.BlockSpec((1, 256), c2),
            pl.BlockSpec((2048, 256), c2),
            pl.BlockSpec((1, 64), c2),
            pl.BlockSpec((1, 64), c2),
            pl.BlockSpec((1, 1), c2),
        ],
        out_specs=pl.BlockSpec((bt, 1), lambda i: (i, 0)),
        scratch_shapes=[pltpu.VMEM((bt, 4096), bf)],
        compiler_params=pltpu.CompilerParams(
            dimension_semantics=("arbitrary",)),
    )(x_t, a1, a2, sh1, sh2, wfc1p, bfc1, wfc2, bfc2)
    return out
